# Initial kernel scaffold; baseline (speedup 1.0000x reference)
import jax
import jax.numpy as jnp
from jax.experimental import pallas as pl


def kernel(query_embeds, passage_bank, passage_tokens, top_k):
    raise NotImplementedError("write your pallas kernel here")


# speedup vs baseline: 3.4200x; 3.4200x over previous
import functools

import jax
import jax.numpy as jnp
from jax import lax
from jax.experimental import pallas as pl
from jax.experimental.pallas import tpu as pltpu
from jax.experimental.pallas import tpu_sc as plsc

_BLK = 8000
_GRP = 40
_K = 16


def _normalize(q):
    ss = jnp.sum(q * q, axis=1, keepdims=True)
    return q / jnp.maximum(jnp.sqrt(ss), 1e-12)


def _stage_a_body(q_ref, bank_ref, gm_ref):
    qn = _normalize(q_ref[...])
    sim = lax.dot_general(
        bank_ref[...], qn, (((1,), (1,)), ((), ())),
        preferred_element_type=jnp.float32)
    ng = sim.shape[0] // _GRP
    gm_ref[...] = jnp.max(sim.reshape(ng, _GRP, sim.shape[1]), axis=1)


def _stage_b_body(gm_ref, out_ref, scr_ref):
    scr_ref[...] = gm_ref[...]
    ng = scr_ref.shape[0]
    ids = lax.broadcasted_iota(jnp.int32, scr_ref.shape, 0)

    def body(k, _):
        g = scr_ref[...]
        m = jnp.max(g, axis=0, keepdims=True)
        sel = jnp.where(g == m, ids, ng)
        amin = jnp.min(sel, axis=0, keepdims=True)
        out_ref[pl.ds(k, 1), :] = amin
        scr_ref[...] = jnp.where(ids == amin, -jnp.inf, g)
        return 0

    lax.fori_loop(0, _K, body, 0)


def _stage_c_body(gids_ref, q_ref, bank_ref, score_ref, fid_ref, acc_ref):
    del gids_ref
    i = pl.program_id(0)
    j = pl.program_id(1)
    qn = _normalize(q_ref[pl.ds(i, 1), :])
    sim = lax.dot_general(
        qn, bank_ref[...], (((1,), (1,)), ((), ())),
        preferred_element_type=jnp.float32)
    acc_ref[pl.ds(j, 1), :] = sim

    @pl.when(j == _K - 1)
    def _():
        s0 = acc_ref[...]
        ids2 = (lax.broadcasted_iota(jnp.int32, s0.shape, 0) * _GRP
                + lax.broadcasted_iota(jnp.int32, s0.shape, 1))
        lane = lax.broadcasted_iota(jnp.int32, (1, _K), 1)

        def body(k, carry):
            s, sv, fv = carry
            m = jnp.max(s)
            f = jnp.min(jnp.where(s == m, ids2, _K * _GRP))
            sv = jnp.where(lane == k, m, sv)
            fv = jnp.where(lane == k, f, fv)
            s = jnp.where(ids2 == f, -jnp.inf, s)
            return s, sv, fv

        _, sv, fv = lax.fori_loop(
            0, _K, body,
            (s0, jnp.zeros((1, _K), jnp.float32), jnp.zeros((1, _K), jnp.int32)))
        score_ref[...] = sv.reshape(1, 1, _K)
        fid_ref[...] = fv.reshape(1, 1, _K)


def _make_sc_token_gather(n_rows, d):
    info = plsc.get_sparse_core_info()
    nw = info.num_cores * info.num_subcores
    bpw = n_rows // nw
    rpg = 128 // d
    assert bpw == info.num_lanes and 128 % d == 0
    mesh = plsc.VectorSubcoreMesh(core_axis_name="c", subcore_axis_name="s")

    @functools.partial(
        pl.kernel, mesh=mesh,
        out_type=jax.ShapeDtypeStruct((n_rows, 128), jnp.int32),
        scratch_types=[
            pltpu.VMEM((bpw,), jnp.int32),
            pltpu.VMEM((bpw,), jnp.int32),
            pltpu.VMEM((bpw, 128), jnp.int32),
            pltpu.VMEM((bpw, d), jnp.int32),
            pltpu.SemaphoreType.DMA,
        ],
    )
    def gather(table_hbm, idx_hbm, out_hbm, idx_v, g_v, gran_v, out_v, sem):
        wid = lax.axis_index("s") * info.num_cores + lax.axis_index("c")
        base = wid * bpw
        pltpu.sync_copy(idx_hbm.at[pl.ds(base, bpw)], idx_v)
        r = idx_v[...]
        shift = rpg.bit_length() - 1
        g_v[...] = lax.shift_right_logical(r, shift)
        pltpu.async_copy(table_hbm.at[g_v], gran_v, sem).wait()
        pltpu.sync_copy(gran_v, out_hbm.at[pl.ds(base, bpw)])

    return gather


def kernel(query_embeds, passage_bank, passage_tokens, top_k):
    n, d = passage_bank.shape
    b = query_embeds.shape[0]
    plen = passage_tokens.shape[1]
    nblk = n // _BLK
    ng = n // _GRP

    gm = pl.pallas_call(
        _stage_a_body,
        grid=(nblk,),
        in_specs=[
            pl.BlockSpec((b, d), lambda g: (0, 0)),
            pl.BlockSpec((_BLK, d), lambda g: (g, 0)),
        ],
        out_specs=pl.BlockSpec((_BLK // _GRP, b), lambda g: (g, 0)),
        out_shape=jax.ShapeDtypeStruct((ng, b), jnp.float32),
    )(query_embeds, passage_bank)

    top_groups = pl.pallas_call(
        _stage_b_body,
        out_shape=jax.ShapeDtypeStruct((_K, b), jnp.int32),
        scratch_shapes=[pltpu.VMEM((ng, b), jnp.float32)],
    )(gm)

    gids = top_groups.T
    grid_spec = pltpu.PrefetchScalarGridSpec(
        num_scalar_prefetch=1,
        grid=(b, _K),
        in_specs=[
            pl.BlockSpec((b, d), lambda i, j, g: (0, 0)),
            pl.BlockSpec((_GRP, d), lambda i, j, g: (g[i * _K + j], 0)),
        ],
        out_specs=[
            pl.BlockSpec((1, 1, _K), lambda i, j, g: (i, 0, 0)),
            pl.BlockSpec((1, 1, _K), lambda i, j, g: (i, 0, 0)),
        ],
        scratch_shapes=[pltpu.VMEM((_K, _GRP), jnp.float32)],
    )
    scores3, fids3 = pl.pallas_call(
        _stage_c_body,
        grid_spec=grid_spec,
        out_shape=[
            jax.ShapeDtypeStruct((b, 1, _K), jnp.float32),
            jax.ShapeDtypeStruct((b, 1, _K), jnp.int32),
        ],
    )(gids.reshape(-1), query_embeds, passage_bank)

    scores = scores3.reshape(b, _K)
    fid = fids3.reshape(b, _K)
    row_idx = jnp.take_along_axis(gids, fid // _GRP, axis=1) * _GRP + fid % _GRP
    row_idx = (row_idx + (top_k - _K)).astype(jnp.int32)

    sc_gather = _make_sc_token_gather(b * _K, plen)
    table128 = passage_tokens.astype(jnp.int32).reshape(n * plen // 128, 128)
    granules = sc_gather(table128, row_idx.reshape(-1))
    off = ((row_idx.reshape(-1) % (128 // plen)) * plen)[:, None] + jnp.arange(plen)[None, :]
    retrieved = jnp.take_along_axis(granules, off, axis=1)
    return retrieved.reshape(b, _K * plen).astype(passage_tokens.dtype), scores


# speedup vs baseline: 4.0511x; 1.1845x over previous
import functools

import jax
import jax.numpy as jnp
from jax import lax
from jax.experimental import pallas as pl
from jax.experimental.pallas import tpu as pltpu
from jax.experimental.pallas import tpu_sc as plsc

_BLK = 8000
_GRP = 40
_K = 16


def _normalize(q):
    ss = jnp.sum(q * q, axis=1, keepdims=True)
    return q / jnp.maximum(jnp.sqrt(ss), 1e-12)


def _stage_a_body(q_ref, bank_ref, gm_ref):
    qn = _normalize(q_ref[...])
    sim = lax.dot_general(
        bank_ref[...], qn, (((1,), (1,)), ((), ())),
        preferred_element_type=jnp.float32)
    ng = sim.shape[0] // _GRP
    gm_ref[...] = jnp.max(sim.reshape(ng, _GRP, sim.shape[1]), axis=1)


def _stage_b_body(gm_ref, out_ref, scr_ref):
    scr_ref[...] = gm_ref[...]
    ng = scr_ref.shape[0]
    ids = lax.broadcasted_iota(jnp.int32, scr_ref.shape, 0)

    def body(k, _):
        g = scr_ref[...]
        m = jnp.max(g, axis=0, keepdims=True)
        sel = jnp.where(g == m, ids, ng)
        amin = jnp.min(sel, axis=0, keepdims=True)
        out_ref[pl.ds(k, 1), :] = amin
        scr_ref[...] = jnp.where(ids == amin, -jnp.inf, g)
        return 0

    lax.fori_loop(0, _K, body, 0)


def _stage_c_body(gids_ref, q_ref, bank_hbm, score_ref, fid_ref, buf, sem):
    i = pl.program_id(0)
    copies = []
    for j in range(_K):
        g = gids_ref[i * _K + j]
        c = pltpu.make_async_copy(
            bank_hbm.at[pl.ds(g * _GRP, _GRP), :],
            buf.at[pl.ds(j * _GRP, _GRP), :], sem)
        c.start()
        copies.append(c)
    for c in copies:
        c.wait()
    qn = _normalize(q_ref[pl.ds(i, 1), :])
    sim = lax.dot_general(
        qn, buf[...], (((1,), (1,)), ((), ())),
        preferred_element_type=jnp.float32)
    ids2 = lax.broadcasted_iota(jnp.int32, sim.shape, 1)
    lane = lax.broadcasted_iota(jnp.int32, (1, _K), 1)

    def body(k, carry):
        s, sv, fv = carry
        m = jnp.max(s)
        f = jnp.min(jnp.where(s == m, ids2, _K * _GRP))
        sv = jnp.where(lane == k, m, sv)
        fv = jnp.where(lane == k, f, fv)
        s = jnp.where(ids2 == f, -jnp.inf, s)
        return s, sv, fv

    _, sv, fv = lax.fori_loop(
        0, _K, body,
        (sim, jnp.zeros((1, _K), jnp.float32), jnp.zeros((1, _K), jnp.int32)))
    score_ref[...] = sv.reshape(1, 1, _K)
    fid_ref[...] = fv.reshape(1, 1, _K)


def _make_sc_token_gather(n_rows, d):
    info = plsc.get_sparse_core_info()
    nw = info.num_cores * info.num_subcores
    bpw = n_rows // nw
    rpg = 128 // d
    assert bpw == info.num_lanes and 128 % d == 0
    mesh = plsc.VectorSubcoreMesh(core_axis_name="c", subcore_axis_name="s")

    @functools.partial(
        pl.kernel, mesh=mesh,
        out_type=jax.ShapeDtypeStruct((n_rows, d), jnp.int32),
        compiler_params=pltpu.CompilerParams(use_tc_tiling_on_sc=False),
        scratch_types=[
            pltpu.VMEM((bpw,), jnp.int32),
            pltpu.VMEM((bpw, d), jnp.int32),
            pltpu.SemaphoreType.DMA,
        ],
    )
    def gather(table_hbm, idx_hbm, out_hbm, idx_v, rows_v, sem):
        wid = lax.axis_index("s") * info.num_cores + lax.axis_index("c")
        base = wid * bpw
        pltpu.sync_copy(idx_hbm.at[pl.ds(base, bpw)], idx_v)
        pltpu.async_copy(table_hbm.at[idx_v], rows_v, sem).wait()
        pltpu.sync_copy(rows_v, out_hbm.at[pl.ds(base, bpw)])

    return gather


def kernel(query_embeds, passage_bank, passage_tokens, top_k):
    n, d = passage_bank.shape
    b = query_embeds.shape[0]
    plen = passage_tokens.shape[1]
    nblk = n // _BLK
    ng = n // _GRP

    gm = pl.pallas_call(
        _stage_a_body,
        grid=(nblk,),
        in_specs=[
            pl.BlockSpec((b, d), lambda g: (0, 0)),
            pl.BlockSpec((_BLK, d), lambda g: (g, 0)),
        ],
        out_specs=pl.BlockSpec((_BLK // _GRP, b), lambda g: (g, 0)),
        out_shape=jax.ShapeDtypeStruct((ng, b), jnp.float32),
    )(query_embeds, passage_bank)

    top_groups = pl.pallas_call(
        _stage_b_body,
        out_shape=jax.ShapeDtypeStruct((_K, b), jnp.int32),
        scratch_shapes=[pltpu.VMEM((ng, b), jnp.float32)],
    )(gm)

    gids = top_groups.T
    grid_spec = pltpu.PrefetchScalarGridSpec(
        num_scalar_prefetch=1,
        grid=(b,),
        in_specs=[
            pl.BlockSpec((b, d), lambda i, g: (0, 0)),
            pl.BlockSpec(memory_space=pl.ANY),
        ],
        out_specs=[
            pl.BlockSpec((1, 1, _K), lambda i, g: (i, 0, 0)),
            pl.BlockSpec((1, 1, _K), lambda i, g: (i, 0, 0)),
        ],
        scratch_shapes=[pltpu.VMEM((_K * _GRP, d), jnp.float32),
                        pltpu.SemaphoreType.DMA],
    )
    scores3, fids3 = pl.pallas_call(
        _stage_c_body,
        grid_spec=grid_spec,
        out_shape=[
            jax.ShapeDtypeStruct((b, 1, _K), jnp.float32),
            jax.ShapeDtypeStruct((b, 1, _K), jnp.int32),
        ],
    )(gids.reshape(-1), query_embeds, passage_bank)

    scores = scores3.reshape(b, _K)
    fid = fids3.reshape(b, _K)
    row_idx = jnp.take_along_axis(gids, fid // _GRP, axis=1) * _GRP + fid % _GRP
    row_idx = (row_idx + (top_k - _K)).astype(jnp.int32)

    sc_gather = _make_sc_token_gather(b * _K, plen)
    retrieved = sc_gather(passage_tokens.astype(jnp.int32), row_idx.reshape(-1))
    return retrieved.reshape(b, _K * plen).astype(passage_tokens.dtype), scores
